# baseline (speedup 1.0000x reference)
import dataclasses

import jax
import jax.numpy as jnp
from jax import lax
from jax.experimental import pallas as pl
from jax.experimental.pallas import tpu as pltpu
from jax.experimental.pallas import tpu_sc as plsc

_NC, _NS, _L = 2, 16, 16
_NW = _NC * _NS
_BATCH = 16384
_D = 64
_BW = _BATCH // _NW
_CH = 128
_NCH = _BW // _CH


def _vsqrt(x):
    i = plsc.bitcast(x, jnp.int32)
    y = plsc.bitcast(jnp.int32(0x5F3759DF) - (i >> 1), jnp.float32)
    for _ in range(3):
        y = y * (1.5 - 0.5 * x * y * y)
    return x * y


def _body(heads_hbm, rels_hbm, tails_hbm, ent_hbm, rel_hbm, out_hbm,
          idx_sp, hidx_s, ridx_s, tidx_s, h_v, r_v, t_v, sq_v, s_v, sem):
    wid = lax.axis_index("s") * _NC + lax.axis_index("c")
    sid = lax.axis_index("s")
    base = wid * _BW
    sb = sid * 3 * _BW
    pltpu.sync_copy(heads_hbm.at[pl.ds(base, _BW)], idx_sp.at[pl.ds(sb, _BW)])
    pltpu.sync_copy(rels_hbm.at[pl.ds(base, _BW)], idx_sp.at[pl.ds(sb + _BW, _BW)])
    pltpu.sync_copy(tails_hbm.at[pl.ds(base, _BW)], idx_sp.at[pl.ds(sb + 2 * _BW, _BW)])
    pltpu.sync_copy(idx_sp.at[pl.ds(sb, _BW)], hidx_s)
    pltpu.sync_copy(idx_sp.at[pl.ds(sb + _BW, _BW)], ridx_s)
    pltpu.sync_copy(idx_sp.at[pl.ds(sb + 2 * _BW, _BW)], tidx_s)

    lanes = lax.iota(jnp.int32, _L)

    @pl.loop(0, _NCH)
    def _chunk(c):
        off = c * _CH

        @pl.loop(0, _CH)
        def _fire(i):
            pltpu.async_copy(ent_hbm.at[pl.ds(hidx_s[off + i], 1)], h_v.at[pl.ds(i, 1)], sem)
            pltpu.async_copy(rel_hbm.at[pl.ds(ridx_s[off + i], 1)], r_v.at[pl.ds(i, 1)], sem)
            pltpu.async_copy(ent_hbm.at[pl.ds(tidx_s[off + i], 1)], t_v.at[pl.ds(i, 1)], sem)

        @pl.loop(0, _CH)
        def _drain(i):
            pltpu.make_async_copy(ent_hbm.at[pl.ds(hidx_s[off + i], 1)], h_v.at[pl.ds(i, 1)], sem).wait()
            pltpu.make_async_copy(rel_hbm.at[pl.ds(ridx_s[off + i], 1)], r_v.at[pl.ds(i, 1)], sem).wait()
            pltpu.make_async_copy(ent_hbm.at[pl.ds(tidx_s[off + i], 1)], t_v.at[pl.ds(i, 1)], sem).wait()

        @pl.loop(0, _CH)
        def _row(i):
            acc = jnp.zeros((_L,), jnp.float32)
            for j in range(_D // _L):
                sl = pl.ds(j * _L, _L)
                d = h_v[i, sl] + r_v[i, sl] - t_v[i, sl]
                acc = acc + d * d
            sq_v[i, :] = acc

        @pl.loop(0, _CH, step=_L)
        def _grp(i0):
            rows = i0 + lanes
            tot = jnp.zeros((_L,), jnp.float32)
            for col in range(_L):
                cols = jnp.full((_L,), col, jnp.int32)
                tot = tot + plsc.load_gather(sq_v, [rows, cols])
            s_v[pl.ds(i0, _L)] = _vsqrt(tot)

        pltpu.sync_copy(s_v, out_hbm.at[pl.ds(base + off, _CH)])


def _tc_transpose(x_t, n_rows):
    blk = 2048
    grid = (n_rows + blk - 1) // blk

    def body(in_ref, out_ref):
        out_ref[...] = in_ref[...].T

    return pl.pallas_call(
        body,
        grid=(grid,),
        in_specs=[pl.BlockSpec((_D, blk), lambda k: (0, k))],
        out_specs=pl.BlockSpec((blk, _D), lambda k: (k, 0)),
        out_shape=jax.ShapeDtypeStruct((n_rows, _D), jnp.float32),
        compiler_params=pltpu.CompilerParams(
            dimension_semantics=("arbitrary",)),
    )(x_t)


@jax.jit
def kernel(heads, relations, tails, entity_emb, relation_emb):
    entity_emb = _tc_transpose(entity_emb.T, entity_emb.shape[0])
    relation_emb = _tc_transpose(relation_emb.T, relation_emb.shape[0])
    mesh = plsc.VectorSubcoreMesh(core_axis_name="c", subcore_axis_name="s")
    cp = pltpu.CompilerParams()
    if "needs_layout_passes" in pltpu.CompilerParams.__dataclass_fields__:
        cp = dataclasses.replace(cp, needs_layout_passes=False)
    run = pl.kernel(
        _body,
        out_type=jax.ShapeDtypeStruct((_BATCH,), jnp.float32),
        mesh=mesh,
        scratch_types=[
            pltpu.VMEM_SHARED((_NS * 3 * _BW,), jnp.int32),
            pltpu.SMEM((_BW,), jnp.int32),
            pltpu.SMEM((_BW,), jnp.int32),
            pltpu.SMEM((_BW,), jnp.int32),
            pltpu.VMEM((_CH, _D), jnp.float32),
            pltpu.VMEM((_CH, _D), jnp.float32),
            pltpu.VMEM((_CH, _D), jnp.float32),
            pltpu.VMEM((_CH, _L), jnp.float32),
            pltpu.VMEM((_CH,), jnp.float32),
            pltpu.SemaphoreType.DMA,
        ],
        compiler_params=cp,
    )
    return run(heads, relations, tails, entity_emb, relation_emb)


# speedup vs baseline: 1.6433x; 1.6433x over previous
import dataclasses

import jax
import jax.numpy as jnp
from jax import lax
from jax.experimental import pallas as pl
from jax.experimental.pallas import tpu as pltpu
from jax.experimental.pallas import tpu_sc as plsc

_NC, _NS, _L = 2, 16, 16
_NW = _NC * _NS
_BATCH = 16384
_D = 64
_BW = _BATCH // _NW
_CH = 128
_NCH = _BW // _CH


def _vsqrt(x):
    i = plsc.bitcast(x, jnp.int32)
    y = plsc.bitcast(jnp.int32(0x5F3759DF) - (i >> 1), jnp.float32)
    for _ in range(3):
        y = y * (1.5 - 0.5 * x * y * y)
    return x * y


def _body(heads_hbm, rels_hbm, tails_hbm, ent_hbm, rel_hbm, out_hbm,
          idx_sp, hidx_s, ridx_s, tidx_s, h_v, r_v, t_v, sq_v, s_v, sem):
    wid = lax.axis_index("s") * _NC + lax.axis_index("c")
    sid = lax.axis_index("s")
    base = wid * _BW
    sb = sid * 3 * _BW
    pltpu.sync_copy(heads_hbm.at[pl.ds(base, _BW)], idx_sp.at[pl.ds(sb, _BW)])
    pltpu.sync_copy(rels_hbm.at[pl.ds(base, _BW)], idx_sp.at[pl.ds(sb + _BW, _BW)])
    pltpu.sync_copy(tails_hbm.at[pl.ds(base, _BW)], idx_sp.at[pl.ds(sb + 2 * _BW, _BW)])
    pltpu.sync_copy(idx_sp.at[pl.ds(sb, _BW)], hidx_s)
    pltpu.sync_copy(idx_sp.at[pl.ds(sb + _BW, _BW)], ridx_s)
    pltpu.sync_copy(idx_sp.at[pl.ds(sb + 2 * _BW, _BW)], tidx_s)

    lanes = lax.iota(jnp.int32, _L)

    @pl.loop(0, _NCH)
    def _chunk(c):
        off = c * _CH

        @pl.loop(0, _CH)
        def _fire(i):
            pltpu.async_copy(ent_hbm.at[pl.ds(hidx_s[off + i], 1)], h_v.at[pl.ds(i, 1)], sem)
            pltpu.async_copy(rel_hbm.at[pl.ds(ridx_s[off + i], 1)], r_v.at[pl.ds(i, 1)], sem)
            pltpu.async_copy(ent_hbm.at[pl.ds(tidx_s[off + i], 1)], t_v.at[pl.ds(i, 1)], sem)

        @pl.loop(0, _CH)
        def _drain(i):
            pltpu.make_async_copy(ent_hbm.at[pl.ds(hidx_s[off + i], 1)], h_v.at[pl.ds(i, 1)], sem).wait()
            pltpu.make_async_copy(rel_hbm.at[pl.ds(ridx_s[off + i], 1)], r_v.at[pl.ds(i, 1)], sem).wait()
            pltpu.make_async_copy(ent_hbm.at[pl.ds(tidx_s[off + i], 1)], t_v.at[pl.ds(i, 1)], sem).wait()

        @pl.loop(0, _CH)
        def _row(i):
            acc = jnp.zeros((_L,), jnp.float32)
            for j in range(_D // _L):
                sl = pl.ds(j * _L, _L)
                d = h_v[i, sl] + r_v[i, sl] - t_v[i, sl]
                acc = acc + d * d
            sq_v[i, :] = acc

        @pl.loop(0, _CH, step=_L)
        def _grp(i0):
            rows = i0 + lanes
            tot = jnp.zeros((_L,), jnp.float32)
            for col in range(_L):
                cols = jnp.full((_L,), col, jnp.int32)
                tot = tot + plsc.load_gather(sq_v, [rows, cols])
            s_v[pl.ds(i0, _L)] = _vsqrt(tot)

        pltpu.sync_copy(s_v, out_hbm.at[pl.ds(base + off, _CH)])


def _tc_transpose(x_t, n_rows):
    blk = 8192
    grid = (n_rows + blk - 1) // blk

    def body(in_ref, out_ref):
        out_ref[...] = in_ref[...].T

    return pl.pallas_call(
        body,
        grid=(grid,),
        in_specs=[pl.BlockSpec((_D, blk), lambda k: (0, k))],
        out_specs=pl.BlockSpec((blk, _D), lambda k: (k, 0)),
        out_shape=jax.ShapeDtypeStruct((n_rows, _D), jnp.float32),
        compiler_params=pltpu.CompilerParams(
            dimension_semantics=("arbitrary",)),
    )(x_t)


@jax.jit
def kernel(heads, relations, tails, entity_emb, relation_emb):
    entity_emb = _tc_transpose(entity_emb.T, entity_emb.shape[0])
    relation_emb = _tc_transpose(relation_emb.T, relation_emb.shape[0])
    mesh = plsc.VectorSubcoreMesh(core_axis_name="c", subcore_axis_name="s")
    cp = pltpu.CompilerParams()
    if "needs_layout_passes" in pltpu.CompilerParams.__dataclass_fields__:
        cp = dataclasses.replace(cp, needs_layout_passes=False)
    run = pl.kernel(
        _body,
        out_type=jax.ShapeDtypeStruct((_BATCH,), jnp.float32),
        mesh=mesh,
        scratch_types=[
            pltpu.VMEM_SHARED((_NS * 3 * _BW,), jnp.int32),
            pltpu.SMEM((_BW,), jnp.int32),
            pltpu.SMEM((_BW,), jnp.int32),
            pltpu.SMEM((_BW,), jnp.int32),
            pltpu.VMEM((_CH, _D), jnp.float32),
            pltpu.VMEM((_CH, _D), jnp.float32),
            pltpu.VMEM((_CH, _D), jnp.float32),
            pltpu.VMEM((_CH, _L), jnp.float32),
            pltpu.VMEM((_CH,), jnp.float32),
            pltpu.SemaphoreType.DMA,
        ],
        compiler_params=cp,
    )
    return run(heads, relations, tails, entity_emb, relation_emb)


# speedup vs baseline: 1.7781x; 1.0820x over previous
import dataclasses

import jax
import jax.numpy as jnp
from jax import lax
from jax.experimental import pallas as pl
from jax.experimental.pallas import tpu as pltpu
from jax.experimental.pallas import tpu_sc as plsc

_NC, _NS, _L = 2, 16, 16
_NW = _NC * _NS
_BATCH = 16384
_D = 64
_BW = _BATCH // _NW
_CH = 128
_NCH = _BW // _CH


def _vsqrt(x):
    i = plsc.bitcast(x, jnp.int32)
    y = plsc.bitcast(jnp.int32(0x5F3759DF) - (i >> 1), jnp.float32)
    for _ in range(3):
        y = y * (1.5 - 0.5 * x * y * y)
    return x * y


def _body(heads_hbm, rels_hbm, tails_hbm, ent_hbm, rel_hbm, out_hbm,
          idx_sp, hidx_s, ridx_s, tidx_s, h_v, r_v, t_v, sq_v, s_v, sem):
    wid = lax.axis_index("s") * _NC + lax.axis_index("c")
    sid = lax.axis_index("s")
    base = wid * _BW
    sb = sid * 3 * _BW
    pltpu.sync_copy(heads_hbm.at[pl.ds(base, _BW)], idx_sp.at[pl.ds(sb, _BW)])
    pltpu.sync_copy(rels_hbm.at[pl.ds(base, _BW)], idx_sp.at[pl.ds(sb + _BW, _BW)])
    pltpu.sync_copy(tails_hbm.at[pl.ds(base, _BW)], idx_sp.at[pl.ds(sb + 2 * _BW, _BW)])
    pltpu.sync_copy(idx_sp.at[pl.ds(sb, _BW)], hidx_s)
    pltpu.sync_copy(idx_sp.at[pl.ds(sb + _BW, _BW)], ridx_s)
    pltpu.sync_copy(idx_sp.at[pl.ds(sb + 2 * _BW, _BW)], tidx_s)

    lanes = lax.iota(jnp.int32, _L)

    @pl.loop(0, _NCH)
    def _chunk(c):
        off = c * _CH

        @pl.loop(0, _CH)
        def _fire(i):
            pltpu.async_copy(ent_hbm.at[pl.ds(hidx_s[off + i], 1)], h_v.at[pl.ds(i, 1)], sem)
            pltpu.async_copy(rel_hbm.at[pl.ds(ridx_s[off + i], 1)], r_v.at[pl.ds(i, 1)], sem)
            pltpu.async_copy(ent_hbm.at[pl.ds(tidx_s[off + i], 1)], t_v.at[pl.ds(i, 1)], sem)

        @pl.loop(0, _CH)
        def _drain(i):
            pltpu.make_async_copy(ent_hbm.at[pl.ds(hidx_s[off + i], 1)], h_v.at[pl.ds(i, 1)], sem).wait()
            pltpu.make_async_copy(rel_hbm.at[pl.ds(ridx_s[off + i], 1)], r_v.at[pl.ds(i, 1)], sem).wait()
            pltpu.make_async_copy(ent_hbm.at[pl.ds(tidx_s[off + i], 1)], t_v.at[pl.ds(i, 1)], sem).wait()

        @pl.loop(0, _CH)
        def _row(i):
            acc = jnp.zeros((_L,), jnp.float32)
            for j in range(_D // _L):
                sl = pl.ds(j * _L, _L)
                d = h_v[i, sl] + r_v[i, sl] - t_v[i, sl]
                acc = acc + d * d
            sq_v[i, :] = acc

        @pl.loop(0, _CH, step=_L)
        def _grp(i0):
            rows = i0 + lanes
            tot = jnp.zeros((_L,), jnp.float32)
            for col in range(_L):
                cols = jnp.full((_L,), col, jnp.int32)
                tot = tot + plsc.load_gather(sq_v, [rows, cols])
            s_v[pl.ds(i0, _L)] = _vsqrt(tot)

        pltpu.sync_copy(s_v, out_hbm.at[pl.ds(base + off, _CH)])


def _tc_transpose(x_t, n_rows):
    blk = 32768
    grid = (n_rows + blk - 1) // blk

    def body(in_ref, out_ref):
        out_ref[...] = in_ref[...].T

    return pl.pallas_call(
        body,
        grid=(grid,),
        in_specs=[pl.BlockSpec((_D, blk), lambda k: (0, k))],
        out_specs=pl.BlockSpec((blk, _D), lambda k: (k, 0)),
        out_shape=jax.ShapeDtypeStruct((n_rows, _D), jnp.float32),
        compiler_params=pltpu.CompilerParams(
            dimension_semantics=("arbitrary",)),
    )(x_t)


@jax.jit
def kernel(heads, relations, tails, entity_emb, relation_emb):
    entity_emb = _tc_transpose(entity_emb.T, entity_emb.shape[0])
    relation_emb = _tc_transpose(relation_emb.T, relation_emb.shape[0])
    mesh = plsc.VectorSubcoreMesh(core_axis_name="c", subcore_axis_name="s")
    cp = pltpu.CompilerParams()
    if "needs_layout_passes" in pltpu.CompilerParams.__dataclass_fields__:
        cp = dataclasses.replace(cp, needs_layout_passes=False)
    run = pl.kernel(
        _body,
        out_type=jax.ShapeDtypeStruct((_BATCH,), jnp.float32),
        mesh=mesh,
        scratch_types=[
            pltpu.VMEM_SHARED((_NS * 3 * _BW,), jnp.int32),
            pltpu.SMEM((_BW,), jnp.int32),
            pltpu.SMEM((_BW,), jnp.int32),
            pltpu.SMEM((_BW,), jnp.int32),
            pltpu.VMEM((_CH, _D), jnp.float32),
            pltpu.VMEM((_CH, _D), jnp.float32),
            pltpu.VMEM((_CH, _D), jnp.float32),
            pltpu.VMEM((_CH, _L), jnp.float32),
            pltpu.VMEM((_CH,), jnp.float32),
            pltpu.SemaphoreType.DMA,
        ],
        compiler_params=cp,
    )
    return run(heads, relations, tails, entity_emb, relation_emb)
